# baseline (speedup 1.0000x reference)
import functools

import jax
import jax.numpy as jnp
from jax import lax
from jax.experimental import pallas as pl
from jax.experimental.pallas import tpu as pltpu
from jax.experimental.pallas import tpu_sc as plsc

N = 10000
E = 320000
CIN = 128
HID = 128
OUT = 10
R = 8
NBASE = 4
G = 64

NC = 2
NS = 16
NW = NC * NS
EPW = E // NW
CH = 80
NCH = EPW // CH
CB = 128
NBCH = EPW // CB
CT = EPW - NBCH * CB
NKEY = N * R
KPT = NKEY // NS


BN = 1000
NBLK = N // BN


def _mesh():
    return plsc.VectorSubcoreMesh(core_axis_name="c", subcore_axis_name="s")


def _sc_counts_body(dst_hbm, rel_hbm, zkey_hbm, ones_hbm, out_hbm,
                    dst_v, rel_v, keys2, ones_v, zbuf, cnt_sp, sem):
    cid = lax.axis_index("c")
    sid = lax.axis_index("s")
    wid = cid * NS + sid
    base = wid * EPW
    pltpu.sync_copy(zkey_hbm, zbuf)
    pltpu.sync_copy(zbuf, cnt_sp.at[pl.ds(sid * KPT, KPT)])
    pltpu.sync_copy(dst_hbm.at[pl.ds(base, EPW)], dst_v)
    pltpu.sync_copy(rel_hbm.at[pl.ds(base, EPW)], rel_v)
    pltpu.sync_copy(ones_hbm, ones_v)

    def mk(c, carry):
        for j in range(CH // 16):
            o = c * CH + j * 16
            keys2[c, pl.ds(j * 16, 16)] = (
                dst_v[pl.ds(o, 16)] * R + rel_v[pl.ds(o, 16)])
        return carry

    lax.fori_loop(0, NCH, mk, 0)
    plsc.subcore_barrier()

    def scat(c, carry):
        pltpu.async_copy(ones_v.at[c], cnt_sp.at[keys2.at[c]], sem, add=True)
        return carry

    lax.fori_loop(0, NCH, scat, 0)

    def drain(c, carry):
        pltpu.make_async_copy(ones_v.at[c], cnt_sp.at[keys2.at[c]], sem).wait()
        return carry

    lax.fori_loop(0, NCH, drain, 0)
    plsc.subcore_barrier()
    pltpu.sync_copy(cnt_sp.at[pl.ds(sid * KPT, KPT)], zbuf)
    pltpu.sync_copy(zbuf, out_hbm.at[pl.ds(cid * NKEY + sid * KPT, KPT)])


@functools.partial(
    pl.kernel,
    out_type=jax.ShapeDtypeStruct((NC * NKEY,), jnp.float32),
    mesh=_mesh(),
    scratch_types=[
        pltpu.VMEM((EPW,), jnp.int32),
        pltpu.VMEM((EPW,), jnp.int32),
        pltpu.VMEM((NCH, CH), jnp.int32),
        pltpu.VMEM((NCH, CH), jnp.float32),
        pltpu.VMEM((KPT,), jnp.float32),
        pltpu.VMEM_SHARED((NKEY,), jnp.float32),
        pltpu.SemaphoreType.DMA,
    ],
)
def _sc_counts(dst_hbm, rel_hbm, zkey_hbm, ones_hbm, out_hbm, *scratch):
    _sc_counts_body(dst_hbm, rel_hbm, zkey_hbm, ones_hbm, out_hbm, *scratch)


def _sc_w_body(dst_hbm, rel_hbm, c0_hbm, c1_hbm, w_hbm,
               dst_v, rel_v, keys2, g0, g1, wv, sem):
    cid = lax.axis_index("c")
    sid = lax.axis_index("s")
    wid = cid * NS + sid
    base = wid * EPW
    pltpu.sync_copy(dst_hbm.at[pl.ds(base, EPW)], dst_v)
    pltpu.sync_copy(rel_hbm.at[pl.ds(base, EPW)], rel_v)

    def mk(c, carry):
        for j in range(CH // 16):
            o = c * CH + j * 16
            keys2[c, pl.ds(j * 16, 16)] = (
                dst_v[pl.ds(o, 16)] * R + rel_v[pl.ds(o, 16)])
        return carry

    lax.fori_loop(0, NCH, mk, 0)

    def gw(c, carry):
        pltpu.async_copy(c0_hbm.at[keys2.at[c]], g0, sem).wait()
        pltpu.async_copy(c1_hbm.at[keys2.at[c]], g1, sem).wait()
        for j in range(CH // 16):
            tot = g0[pl.ds(j * 16, 16)] + g1[pl.ds(j * 16, 16)]
            wv[pl.ds(c * CH + j * 16, 16)] = 1.0 / tot
        return carry

    lax.fori_loop(0, NCH, gw, 0)
    pltpu.sync_copy(wv, w_hbm.at[pl.ds(base, EPW)])


@functools.partial(
    pl.kernel,
    out_type=jax.ShapeDtypeStruct((E,), jnp.float32),
    mesh=_mesh(),
    scratch_types=[
        pltpu.VMEM((EPW,), jnp.int32),
        pltpu.VMEM((EPW,), jnp.int32),
        pltpu.VMEM((NCH, CH), jnp.int32),
        pltpu.VMEM((CH,), jnp.float32),
        pltpu.VMEM((CH,), jnp.float32),
        pltpu.VMEM((EPW,), jnp.float32),
        pltpu.SemaphoreType.DMA,
    ],
)
def _sc_w(dst_hbm, rel_hbm, c0_hbm, c1_hbm, w_hbm, *scratch):
    _sc_w_body(dst_hbm, rel_hbm, c0_hbm, c1_hbm, w_hbm, *scratch)


HH = HID // 2


def _sc_layer_body(xw_hbm, src_hbm, rel_hbm, dst_hbm, w_hbm, zrow_hbm, out_hbm,
                   src_v, rel_v, dst_v, g2, g2f, g2t, d2, d2t, wv, rows,
                   rowsb, agg_sp, sem, semb):
    cid = lax.axis_index("c")
    sid = lax.axis_index("s")
    wid = cid * NS + sid
    base = wid * EPW
    pltpu.sync_copy(src_hbm.at[pl.ds(base, EPW)], src_v)
    pltpu.sync_copy(rel_hbm.at[pl.ds(base, EPW)], rel_v)
    pltpu.sync_copy(dst_hbm.at[pl.ds(base, EPW)], dst_v)
    pltpu.sync_copy(w_hbm.at[pl.ds(base, EPW)], wv)

    def mk(c, carry):
        for j in range(CB // 16):
            o = c * CB + j * 16
            g2[c, pl.ds(j * 16, 16)] = (
                src_v[pl.ds(o, 16)] * (2 * R) + rel_v[pl.ds(o, 16)] * 2)
            d2[c, pl.ds(j * 16, 16)] = dst_v[pl.ds(o, 16)]
        return carry

    lax.fori_loop(0, NBCH, mk, 0)
    ot = NBCH * CB
    g2t[...] = src_v[pl.ds(ot, CT)] * (2 * R) + rel_v[pl.ds(ot, CT)] * 2
    d2t[...] = dst_v[pl.ds(ot, CT)]
    pltpu.sync_copy(zrow_hbm, rows)

    def do_scale(buf, c):
        def tstep(t, carry):
            wl = wv[pl.ds(c * CB + t * 16, 16)]
            for e16 in range(16):
                ws = wl[e16]
                for q in range(HH // 16):
                    buf[t * 16 + e16, pl.ds(q * 16, 16)] = (
                        buf[t * 16 + e16, pl.ds(q * 16, 16)] * ws)
            return carry

        lax.fori_loop(0, CB // 16, tstep, 0)

    def do_scale_tail(buf):
        wl = wv[pl.ds(NBCH * CB, CT)]
        for e16 in range(CT):
            ws = wl[e16]
            for q in range(HH // 16):
                buf[e16, pl.ds(q * 16, 16)] = buf[e16, pl.ds(q * 16, 16)] * ws

    for f in (0, 1):
        def zloop(j, carry):
            ck = sid + j * NS

            @pl.when(ck < N // CH)
            def _():
                pltpu.sync_copy(rows.at[pl.ds(0, CH)],
                                agg_sp.at[pl.ds(ck * CH, CH)])

            return carry

        lax.fori_loop(0, N // CH // NS + 1, zloop, 0)

        def mkf(c, carry):
            for j in range(CB // 16):
                g2f[c, pl.ds(j * 16, 16)] = g2[c, pl.ds(j * 16, 16)] + f
            return carry

        lax.fori_loop(0, NBCH, mkf, 0)
        if f == 1:
            g2t[...] = g2t[...] + 1
        plsc.subcore_barrier()

        def work(c, carry):
            pltpu.async_copy(xw_hbm.at[g2f.at[c]], rows, sem).wait()
            do_scale(rows, c)
            pltpu.sync_copy(rows, agg_sp.at[d2.at[c]], add=True)
            return carry

        lax.fori_loop(0, NBCH, work, 0)
        pltpu.async_copy(xw_hbm.at[g2t], rowsb, semb).wait()
        do_scale_tail(rowsb)
        pltpu.sync_copy(rowsb, agg_sp.at[d2t], add=True)
        plsc.subcore_barrier()

        def rloop(j, carry):
            ck = sid + j * NS

            @pl.when(ck < N // CH)
            def _():
                pltpu.sync_copy(agg_sp.at[pl.ds(ck * CH, CH)],
                                rows.at[pl.ds(0, CH)])
                pltpu.sync_copy(
                    rows.at[pl.ds(0, CH)],
                    out_hbm.at[pl.ds(f * NC * N + cid * N + ck * CH, CH)])

            return carry

        lax.fori_loop(0, N // CH // NS + 1, rloop, 0)
        if f == 0:
            plsc.subcore_barrier()
        pltpu.sync_copy(zrow_hbm, rows)


@functools.partial(
    pl.kernel,
    out_type=jax.ShapeDtypeStruct((2 * NC * N, HH), jnp.float32),
    mesh=_mesh(),
    compiler_params=pltpu.CompilerParams(use_tc_tiling_on_sc=False),
    scratch_types=[
        pltpu.VMEM((EPW,), jnp.int32),
        pltpu.VMEM((EPW,), jnp.int32),
        pltpu.VMEM((EPW,), jnp.int32),
        pltpu.VMEM((NBCH, CB), jnp.int32),
        pltpu.VMEM((NBCH, CB), jnp.int32),
        pltpu.VMEM((CT,), jnp.int32),
        pltpu.VMEM((NBCH, CB), jnp.int32),
        pltpu.VMEM((CT,), jnp.int32),
        pltpu.VMEM((EPW,), jnp.float32),
        pltpu.VMEM((CB, HH), jnp.float32),
        pltpu.VMEM((CT, HH), jnp.float32),
        pltpu.VMEM_SHARED((N, HH), jnp.float32),
        pltpu.SemaphoreType.DMA,
        pltpu.SemaphoreType.DMA,
    ],
)
def _sc_layer(xw_hbm, src_hbm, rel_hbm, dst_hbm, w_hbm, zrow_hbm, out_hbm,
              *scratch):
    _sc_layer_body(xw_hbm, src_hbm, rel_hbm, dst_hbm, w_hbm, zrow_hbm, out_hbm,
                   *scratch)


def _kw_body(comp1_ref, b1_ref, comp2_ref, b2_ref, w1_ref, w2_ref):
    for cref, bref, oref in ((comp1_ref, b1_ref, w1_ref),
                             (comp2_ref, b2_ref, w2_ref)):
        for r in range(R):
            acc = cref[r, 0] * bref[0]
            for b in range(1, NBASE):
                acc = acc + cref[r, b] * bref[b]
            oref[:, r * HID:(r + 1) * HID] = acc


def _kw(comp1, bases1, comp2, bases2):
    return pl.pallas_call(
        _kw_body,
        in_specs=[
            pl.BlockSpec(memory_space=pltpu.SMEM),
            pl.BlockSpec(memory_space=pltpu.MemorySpace.VMEM),
            pl.BlockSpec(memory_space=pltpu.SMEM),
            pl.BlockSpec(memory_space=pltpu.MemorySpace.VMEM),
        ],
        out_specs=[pl.BlockSpec(memory_space=pltpu.MemorySpace.VMEM),
                   pl.BlockSpec(memory_space=pltpu.MemorySpace.VMEM)],
        out_shape=[jax.ShapeDtypeStruct((CIN, R * HID), jnp.float32),
                   jax.ShapeDtypeStruct((HID, R * HID), jnp.float32)],
    )(comp1, bases1, comp2, bases2)


def _dense_body(x_ref, w_ref, o_ref):
    o_ref[...] = jnp.dot(x_ref[...], w_ref[...],
                         preferred_element_type=jnp.float32)


def _dense(x, w_all):
    return pl.pallas_call(
        _dense_body,
        grid=(NBLK,),
        in_specs=[
            pl.BlockSpec((BN, CIN), lambda i: (i, 0)),
            pl.BlockSpec((CIN, R * HID), lambda i: (0, 0)),
        ],
        out_specs=pl.BlockSpec((BN, R * HID), lambda i: (i, 0)),
        out_shape=jax.ShapeDtypeStruct((N, R * HID), jnp.float32),
    )(x, w_all)


def _comb_body(p_ref, x_ref, root_ref,
               bias_ref, w2_ref, h_ref, o_ref):
    agg = jnp.concatenate([p_ref[0] + p_ref[1],
                           p_ref[2] + p_ref[3]], axis=1)
    h = (agg
         + jnp.dot(x_ref[...], root_ref[...],
                   preferred_element_type=jnp.float32)
         + bias_ref[...])
    h = jnp.maximum(h, 0.0)
    h_ref[...] = h
    o_ref[...] = jnp.dot(h, w2_ref[...], preferred_element_type=jnp.float32)


def _comb(p4, x, root, bias_row, w2_all):
    return pl.pallas_call(
        _comb_body,
        grid=(NBLK,),
        in_specs=[
            pl.BlockSpec((4, BN, HH), lambda i: (0, i, 0)),
            pl.BlockSpec((BN, CIN), lambda i: (i, 0)),
            pl.BlockSpec((CIN, HID), lambda i: (0, 0)),
            pl.BlockSpec((1, HID), lambda i: (0, 0)),
            pl.BlockSpec((HID, R * HID), lambda i: (0, 0)),
        ],
        out_specs=[pl.BlockSpec((BN, HID), lambda i: (i, 0)),
                   pl.BlockSpec((BN, R * HID), lambda i: (i, 0))],
        out_shape=[jax.ShapeDtypeStruct((N, HID), jnp.float32),
                   jax.ShapeDtypeStruct((N, R * HID), jnp.float32)],
    )(p4, x, root, bias_row, w2_all)


def _final_body(q_ref, h1_ref, root_ref,
                bias_ref, batch_ref, wc_ref, bc_ref, o_ref, acc_ref, cnt_ref):
    i = pl.program_id(0)

    @pl.when(i == 0)
    def _init():
        acc_ref[...] = jnp.zeros_like(acc_ref)
        cnt_ref[...] = jnp.zeros_like(cnt_ref)

    agg = jnp.concatenate([q_ref[0] + q_ref[1],
                           q_ref[2] + q_ref[3]], axis=1)
    h2 = (agg
          + jnp.dot(h1_ref[...], root_ref[...],
                    preferred_element_type=jnp.float32)
          + bias_ref[...])
    h2 = jnp.maximum(h2, 0.0)
    b = batch_ref[0, 0, :]
    gi = lax.broadcasted_iota(jnp.int32, (G, BN), 0)
    mask = (gi == b[None, :]).astype(jnp.float32)
    acc_ref[...] += jnp.dot(mask, h2, preferred_element_type=jnp.float32)
    cnt_ref[...] += jnp.sum(mask, axis=1, keepdims=True)

    @pl.when(i == NBLK - 1)
    def _fin():
        pooled = acc_ref[...] / jnp.maximum(cnt_ref[...], 1.0)
        o_ref[...] = (jnp.dot(pooled, wc_ref[...],
                              preferred_element_type=jnp.float32)
                      + bc_ref[...])


def _final(q4, h1, root2, bias2_row, batch3, Wc, bc_row):
    return pl.pallas_call(
        _final_body,
        grid=(NBLK,),
        in_specs=[
            pl.BlockSpec((4, BN, HH), lambda i: (0, i, 0)),
            pl.BlockSpec((BN, HID), lambda i: (i, 0)),
            pl.BlockSpec((HID, HID), lambda i: (0, 0)),
            pl.BlockSpec((1, HID), lambda i: (0, 0)),
            pl.BlockSpec((1, 1, BN), lambda i: (i, 0, 0)),
            pl.BlockSpec((HID, OUT), lambda i: (0, 0)),
            pl.BlockSpec((1, OUT), lambda i: (0, 0)),
        ],
        out_specs=pl.BlockSpec((G, OUT), lambda i: (0, 0)),
        out_shape=jax.ShapeDtypeStruct((G, OUT), jnp.float32),
        scratch_shapes=[pltpu.VMEM((G, HID), jnp.float32),
                        pltpu.VMEM((G, 1), jnp.float32)],
    )(q4, h1, root2, bias2_row, batch3, Wc, bc_row)


def kernel(x, edge_index, edge_type, batch, bases1, comp1, root1, bias1,
           bases2, comp2, root2, bias2, Wc, bc):
    src = edge_index[0]
    dst = edge_index[1]
    rel = edge_type
    batch3 = batch.reshape(NBLK, 1, BN)
    zkey = jnp.zeros((KPT,), jnp.float32)
    zrow = jnp.zeros((CB, HH), jnp.float32)
    bias1r = bias1.reshape(1, HID)
    bias2r = bias2.reshape(1, HID)
    bcr = bc.reshape(1, OUT)

    w1a, w2a = _kw(comp1, bases1, comp2, bases2)
    ones2 = jnp.ones((NCH, CH), jnp.float32)
    cnts = _sc_counts(dst, rel, zkey, ones2)
    wvec = _sc_w(dst, rel, cnts[:NKEY], cnts[NKEY:])
    xw1 = _dense(x, w1a).reshape(2 * NKEY, HH)
    a1 = _sc_layer(xw1, src, rel, dst, wvec, zrow).reshape(4, N, HH)
    h1, xw2 = _comb(a1, x, root1, bias1r, w2a)
    a2 = _sc_layer(xw2.reshape(2 * NKEY, HH), src, rel, dst, wvec,
                   zrow).reshape(4, N, HH)
    return _final(a2, h1, root2, bias2r, batch3, Wc, bcr)


# speedup vs baseline: 1.3857x; 1.3857x over previous
import functools

import jax
import jax.numpy as jnp
from jax import lax
from jax.experimental import pallas as pl
from jax.experimental.pallas import tpu as pltpu
from jax.experimental.pallas import tpu_sc as plsc

N = 10000
E = 320000
CIN = 128
HID = 128
OUT = 10
R = 8
NBASE = 4
G = 64

NC = 2
NS = 16
NW = NC * NS
EPW = E // NW
CH = 80
NCH = EPW // CH
NKEY = N * R
KPT = NKEY // NS


BN = 1000
NBLK = N // BN


def _mesh():
    return plsc.VectorSubcoreMesh(core_axis_name="c", subcore_axis_name="s")


def _sc_counts_body(dst_hbm, rel_hbm, zkey_hbm, ones_hbm, out_hbm,
                    dst_v, rel_v, keys2, ones_v, zbuf, cnt_sp, sem):
    cid = lax.axis_index("c")
    sid = lax.axis_index("s")
    wid = cid * NS + sid
    base = wid * EPW
    pltpu.sync_copy(zkey_hbm, zbuf)
    pltpu.sync_copy(zbuf, cnt_sp.at[pl.ds(sid * KPT, KPT)])
    pltpu.sync_copy(dst_hbm.at[pl.ds(base, EPW)], dst_v)
    pltpu.sync_copy(rel_hbm.at[pl.ds(base, EPW)], rel_v)
    pltpu.sync_copy(ones_hbm, ones_v)

    def mk(c, carry):
        for j in range(CH // 16):
            o = c * CH + j * 16
            keys2[c, pl.ds(j * 16, 16)] = (
                dst_v[pl.ds(o, 16)] * R + rel_v[pl.ds(o, 16)])
        return carry

    lax.fori_loop(0, NCH, mk, 0)
    plsc.subcore_barrier()

    def scat(c, carry):
        pltpu.async_copy(ones_v.at[c], cnt_sp.at[keys2.at[c]], sem, add=True)
        return carry

    lax.fori_loop(0, NCH, scat, 0)

    def drain(c, carry):
        pltpu.make_async_copy(ones_v.at[c], cnt_sp.at[keys2.at[c]], sem).wait()
        return carry

    lax.fori_loop(0, NCH, drain, 0)
    plsc.subcore_barrier()
    pltpu.sync_copy(cnt_sp.at[pl.ds(sid * KPT, KPT)], zbuf)
    pltpu.sync_copy(zbuf, out_hbm.at[pl.ds(cid * NKEY + sid * KPT, KPT)])


@functools.partial(
    pl.kernel,
    out_type=jax.ShapeDtypeStruct((NC * NKEY,), jnp.float32),
    mesh=_mesh(),
    scratch_types=[
        pltpu.VMEM((EPW,), jnp.int32),
        pltpu.VMEM((EPW,), jnp.int32),
        pltpu.VMEM((NCH, CH), jnp.int32),
        pltpu.VMEM((NCH, CH), jnp.float32),
        pltpu.VMEM((KPT,), jnp.float32),
        pltpu.VMEM_SHARED((NKEY,), jnp.float32),
        pltpu.SemaphoreType.DMA,
    ],
)
def _sc_counts(dst_hbm, rel_hbm, zkey_hbm, ones_hbm, out_hbm, *scratch):
    _sc_counts_body(dst_hbm, rel_hbm, zkey_hbm, ones_hbm, out_hbm, *scratch)


def _sc_w_body(dst_hbm, rel_hbm, c0_hbm, c1_hbm, w_hbm,
               dst_v, rel_v, keys2, g0, g1, wv, sem):
    cid = lax.axis_index("c")
    sid = lax.axis_index("s")
    wid = cid * NS + sid
    base = wid * EPW
    pltpu.sync_copy(dst_hbm.at[pl.ds(base, EPW)], dst_v)
    pltpu.sync_copy(rel_hbm.at[pl.ds(base, EPW)], rel_v)

    def mk(c, carry):
        for j in range(CH // 16):
            o = c * CH + j * 16
            keys2[c, pl.ds(j * 16, 16)] = (
                dst_v[pl.ds(o, 16)] * R + rel_v[pl.ds(o, 16)])
        return carry

    lax.fori_loop(0, NCH, mk, 0)

    def gw(c, carry):
        pltpu.async_copy(c0_hbm.at[keys2.at[c]], g0, sem).wait()
        pltpu.async_copy(c1_hbm.at[keys2.at[c]], g1, sem).wait()
        for j in range(CH // 16):
            tot = g0[pl.ds(j * 16, 16)] + g1[pl.ds(j * 16, 16)]
            wv[pl.ds(c * CH + j * 16, 16)] = 1.0 / tot
        return carry

    lax.fori_loop(0, NCH, gw, 0)
    pltpu.sync_copy(wv, w_hbm.at[pl.ds(base, EPW)])


@functools.partial(
    pl.kernel,
    out_type=jax.ShapeDtypeStruct((E,), jnp.float32),
    mesh=_mesh(),
    scratch_types=[
        pltpu.VMEM((EPW,), jnp.int32),
        pltpu.VMEM((EPW,), jnp.int32),
        pltpu.VMEM((NCH, CH), jnp.int32),
        pltpu.VMEM((CH,), jnp.float32),
        pltpu.VMEM((CH,), jnp.float32),
        pltpu.VMEM((EPW,), jnp.float32),
        pltpu.SemaphoreType.DMA,
    ],
)
def _sc_w(dst_hbm, rel_hbm, c0_hbm, c1_hbm, w_hbm, *scratch):
    _sc_w_body(dst_hbm, rel_hbm, c0_hbm, c1_hbm, w_hbm, *scratch)


HH = HID // 2


def _sc_layer_body(xw_hbm, src_hbm, rel_hbm, dst_hbm, w_hbm, zrow_hbm, out_hbm,
                   src_v, rel_v, dst_v, g2, g2f, d2, wv, rows, agg_sp, sem):
    cid = lax.axis_index("c")
    sid = lax.axis_index("s")
    wid = cid * NS + sid
    base = wid * EPW
    pltpu.sync_copy(src_hbm.at[pl.ds(base, EPW)], src_v)
    pltpu.sync_copy(rel_hbm.at[pl.ds(base, EPW)], rel_v)
    pltpu.sync_copy(dst_hbm.at[pl.ds(base, EPW)], dst_v)
    pltpu.sync_copy(w_hbm.at[pl.ds(base, EPW)], wv)

    def mk(c, carry):
        for j in range(CH // 16):
            o = c * CH + j * 16
            g2[c, pl.ds(j * 16, 16)] = (
                src_v[pl.ds(o, 16)] * (2 * R) + rel_v[pl.ds(o, 16)] * 2)
            d2[c, pl.ds(j * 16, 16)] = dst_v[pl.ds(o, 16)]
        return carry

    lax.fori_loop(0, NCH, mk, 0)
    pltpu.sync_copy(zrow_hbm, rows)

    for f in (0, 1):
        def zloop(j, carry):
            ck = sid + j * NS

            @pl.when(ck < N // CH)
            def _():
                pltpu.sync_copy(rows, agg_sp.at[pl.ds(ck * CH, CH)])

            return carry

        lax.fori_loop(0, N // CH // NS + 1, zloop, 0)

        def mkf(c, carry):
            for j in range(CH // 16):
                g2f[c, pl.ds(j * 16, 16)] = g2[c, pl.ds(j * 16, 16)] + f
            return carry

        lax.fori_loop(0, NCH, mkf, 0)
        plsc.subcore_barrier()

        def work(c, carry):
            pltpu.async_copy(xw_hbm.at[g2f.at[c]], rows, sem).wait()
            for t in range(CH // 16):
                wl = wv[pl.ds(c * CH + t * 16, 16)]
                for e16 in range(16):
                    e = t * 16 + e16
                    ws = wl[e16]
                    for q in range(HH // 16):
                        rows[e, pl.ds(q * 16, 16)] = (
                            rows[e, pl.ds(q * 16, 16)] * ws)
            pltpu.sync_copy(rows, agg_sp.at[d2.at[c]], add=True)
            return carry

        lax.fori_loop(0, NCH, work, 0)
        plsc.subcore_barrier()

        def rloop(j, carry):
            ck = sid + j * NS

            @pl.when(ck < N // CH)
            def _():
                pltpu.sync_copy(agg_sp.at[pl.ds(ck * CH, CH)], rows)
                pltpu.sync_copy(
                    rows,
                    out_hbm.at[pl.ds(f * NC * N + cid * N + ck * CH, CH)])

            return carry

        lax.fori_loop(0, N // CH // NS + 1, rloop, 0)
        if f == 0:
            plsc.subcore_barrier()
        pltpu.sync_copy(zrow_hbm, rows)


@functools.partial(
    pl.kernel,
    out_type=jax.ShapeDtypeStruct((2 * NC * N, HH), jnp.float32),
    mesh=_mesh(),
    compiler_params=pltpu.CompilerParams(use_tc_tiling_on_sc=False),
    scratch_types=[
        pltpu.VMEM((EPW,), jnp.int32),
        pltpu.VMEM((EPW,), jnp.int32),
        pltpu.VMEM((EPW,), jnp.int32),
        pltpu.VMEM((NCH, CH), jnp.int32),
        pltpu.VMEM((NCH, CH), jnp.int32),
        pltpu.VMEM((NCH, CH), jnp.int32),
        pltpu.VMEM((EPW,), jnp.float32),
        pltpu.VMEM((CH, HH), jnp.float32),
        pltpu.VMEM_SHARED((N, HH), jnp.float32),
        pltpu.SemaphoreType.DMA,
    ],
)
def _sc_layer(xw_hbm, src_hbm, rel_hbm, dst_hbm, w_hbm, zrow_hbm, out_hbm,
              *scratch):
    _sc_layer_body(xw_hbm, src_hbm, rel_hbm, dst_hbm, w_hbm, zrow_hbm, out_hbm,
                   *scratch)


def _kw_body(comp1_ref, b1_ref, comp2_ref, b2_ref, w1_ref, w2_ref):
    for cref, bref, oref in ((comp1_ref, b1_ref, w1_ref),
                             (comp2_ref, b2_ref, w2_ref)):
        for r in range(R):
            acc = cref[r, 0] * bref[0]
            for b in range(1, NBASE):
                acc = acc + cref[r, b] * bref[b]
            oref[:, r * HID:(r + 1) * HID] = acc


def _kw(comp1, bases1, comp2, bases2):
    return pl.pallas_call(
        _kw_body,
        in_specs=[
            pl.BlockSpec(memory_space=pltpu.SMEM),
            pl.BlockSpec(memory_space=pltpu.MemorySpace.VMEM),
            pl.BlockSpec(memory_space=pltpu.SMEM),
            pl.BlockSpec(memory_space=pltpu.MemorySpace.VMEM),
        ],
        out_specs=[pl.BlockSpec(memory_space=pltpu.MemorySpace.VMEM),
                   pl.BlockSpec(memory_space=pltpu.MemorySpace.VMEM)],
        out_shape=[jax.ShapeDtypeStruct((CIN, R * HID), jnp.float32),
                   jax.ShapeDtypeStruct((HID, R * HID), jnp.float32)],
    )(comp1, bases1, comp2, bases2)


def _dense_body(x_ref, w_ref, o_ref):
    o_ref[...] = jnp.dot(x_ref[...], w_ref[...],
                         preferred_element_type=jnp.float32)


def _dense(x, w_all):
    return pl.pallas_call(
        _dense_body,
        grid=(NBLK,),
        in_specs=[
            pl.BlockSpec((BN, CIN), lambda i: (i, 0)),
            pl.BlockSpec((CIN, R * HID), lambda i: (0, 0)),
        ],
        out_specs=pl.BlockSpec((BN, R * HID), lambda i: (i, 0)),
        out_shape=jax.ShapeDtypeStruct((N, R * HID), jnp.float32),
    )(x, w_all)


def _comb_body(p_ref, x_ref, root_ref,
               bias_ref, w2_ref, h_ref, o_ref):
    agg = jnp.concatenate([p_ref[0] + p_ref[1],
                           p_ref[2] + p_ref[3]], axis=1)
    h = (agg
         + jnp.dot(x_ref[...], root_ref[...],
                   preferred_element_type=jnp.float32)
         + bias_ref[...])
    h = jnp.maximum(h, 0.0)
    h_ref[...] = h
    o_ref[...] = jnp.dot(h, w2_ref[...], preferred_element_type=jnp.float32)


def _comb(p4, x, root, bias_row, w2_all):
    return pl.pallas_call(
        _comb_body,
        grid=(NBLK,),
        in_specs=[
            pl.BlockSpec((4, BN, HH), lambda i: (0, i, 0)),
            pl.BlockSpec((BN, CIN), lambda i: (i, 0)),
            pl.BlockSpec((CIN, HID), lambda i: (0, 0)),
            pl.BlockSpec((1, HID), lambda i: (0, 0)),
            pl.BlockSpec((HID, R * HID), lambda i: (0, 0)),
        ],
        out_specs=[pl.BlockSpec((BN, HID), lambda i: (i, 0)),
                   pl.BlockSpec((BN, R * HID), lambda i: (i, 0))],
        out_shape=[jax.ShapeDtypeStruct((N, HID), jnp.float32),
                   jax.ShapeDtypeStruct((N, R * HID), jnp.float32)],
    )(p4, x, root, bias_row, w2_all)


def _final_body(q_ref, h1_ref, root_ref,
                bias_ref, batch_ref, wc_ref, bc_ref, o_ref, acc_ref, cnt_ref):
    i = pl.program_id(0)

    @pl.when(i == 0)
    def _init():
        acc_ref[...] = jnp.zeros_like(acc_ref)
        cnt_ref[...] = jnp.zeros_like(cnt_ref)

    agg = jnp.concatenate([q_ref[0] + q_ref[1],
                           q_ref[2] + q_ref[3]], axis=1)
    h2 = (agg
          + jnp.dot(h1_ref[...], root_ref[...],
                    preferred_element_type=jnp.float32)
          + bias_ref[...])
    h2 = jnp.maximum(h2, 0.0)
    b = batch_ref[0, 0, :]
    gi = lax.broadcasted_iota(jnp.int32, (G, BN), 0)
    mask = (gi == b[None, :]).astype(jnp.float32)
    acc_ref[...] += jnp.dot(mask, h2, preferred_element_type=jnp.float32)
    cnt_ref[...] += jnp.sum(mask, axis=1, keepdims=True)

    @pl.when(i == NBLK - 1)
    def _fin():
        pooled = acc_ref[...] / jnp.maximum(cnt_ref[...], 1.0)
        o_ref[...] = (jnp.dot(pooled, wc_ref[...],
                              preferred_element_type=jnp.float32)
                      + bc_ref[...])


def _final(q4, h1, root2, bias2_row, batch3, Wc, bc_row):
    return pl.pallas_call(
        _final_body,
        grid=(NBLK,),
        in_specs=[
            pl.BlockSpec((4, BN, HH), lambda i: (0, i, 0)),
            pl.BlockSpec((BN, HID), lambda i: (i, 0)),
            pl.BlockSpec((HID, HID), lambda i: (0, 0)),
            pl.BlockSpec((1, HID), lambda i: (0, 0)),
            pl.BlockSpec((1, 1, BN), lambda i: (i, 0, 0)),
            pl.BlockSpec((HID, OUT), lambda i: (0, 0)),
            pl.BlockSpec((1, OUT), lambda i: (0, 0)),
        ],
        out_specs=pl.BlockSpec((G, OUT), lambda i: (0, 0)),
        out_shape=jax.ShapeDtypeStruct((G, OUT), jnp.float32),
        scratch_shapes=[pltpu.VMEM((G, HID), jnp.float32),
                        pltpu.VMEM((G, 1), jnp.float32)],
    )(q4, h1, root2, bias2_row, batch3, Wc, bc_row)


def kernel(x, edge_index, edge_type, batch, bases1, comp1, root1, bias1,
           bases2, comp2, root2, bias2, Wc, bc):
    src = edge_index[0]
    dst = edge_index[1]
    rel = edge_type
    batch3 = batch.reshape(NBLK, 1, BN)
    zkey = jnp.zeros((KPT,), jnp.float32)
    zrow = jnp.zeros((CH, HH), jnp.float32)
    bias1r = bias1.reshape(1, HID)
    bias2r = bias2.reshape(1, HID)
    bcr = bc.reshape(1, OUT)

    w1a, w2a = _kw(comp1, bases1, comp2, bases2)
    ones2 = jnp.ones((NCH, CH), jnp.float32)
    cnts = _sc_counts(dst, rel, zkey, ones2)
    wvec = _sc_w(dst, rel, cnts[:NKEY], cnts[NKEY:])
    xw1 = _dense(x, w1a).reshape(2 * NKEY, HH)
    a1 = _sc_layer(xw1, src, rel, dst, wvec, zrow).reshape(4, N, HH)
    h1, xw2 = _comb(a1, x, root1, bias1r, w2a)
    a2 = _sc_layer(xw2.reshape(2 * NKEY, HH), src, rel, dst, wvec,
                   zrow).reshape(4, N, HH)
    return _final(a2, h1, root2, bias2r, batch3, Wc, bcr)


# speedup vs baseline: 1.6117x; 1.1631x over previous
import functools

import jax
import jax.numpy as jnp
from jax import lax
from jax.experimental import pallas as pl
from jax.experimental.pallas import tpu as pltpu
from jax.experimental.pallas import tpu_sc as plsc

N = 10000
E = 320000
CIN = 128
HID = 128
OUT = 10
R = 8
NBASE = 4
G = 64

NC = 2
NS = 16
NW = NC * NS
EPW = E // NW
CH = 80
NCH = EPW // CH
NKEY = N * R
KPT = NKEY // NS


BN = 1000
NBLK = N // BN


def _mesh():
    return plsc.VectorSubcoreMesh(core_axis_name="c", subcore_axis_name="s")


def _sc_counts_body(dst_hbm, rel_hbm, zkey_hbm, ones_hbm, out_hbm,
                    dst_v, rel_v, keys2, ones_v, zbuf, cnt_sp, sem):
    cid = lax.axis_index("c")
    sid = lax.axis_index("s")
    wid = cid * NS + sid
    base = wid * EPW
    pltpu.sync_copy(zkey_hbm, zbuf)
    pltpu.sync_copy(zbuf, cnt_sp.at[pl.ds(sid * KPT, KPT)])
    pltpu.sync_copy(dst_hbm.at[pl.ds(base, EPW)], dst_v)
    pltpu.sync_copy(rel_hbm.at[pl.ds(base, EPW)], rel_v)
    pltpu.sync_copy(ones_hbm, ones_v)

    def mk(c, carry):
        for j in range(CH // 16):
            o = c * CH + j * 16
            keys2[c, pl.ds(j * 16, 16)] = (
                dst_v[pl.ds(o, 16)] * R + rel_v[pl.ds(o, 16)])
        return carry

    lax.fori_loop(0, NCH, mk, 0)
    plsc.subcore_barrier()

    def scat(c, carry):
        pltpu.async_copy(ones_v.at[c], cnt_sp.at[keys2.at[c]], sem, add=True)
        return carry

    lax.fori_loop(0, NCH, scat, 0)

    def drain(c, carry):
        pltpu.make_async_copy(ones_v.at[c], cnt_sp.at[keys2.at[c]], sem).wait()
        return carry

    lax.fori_loop(0, NCH, drain, 0)
    plsc.subcore_barrier()
    pltpu.sync_copy(cnt_sp.at[pl.ds(sid * KPT, KPT)], zbuf)
    pltpu.sync_copy(zbuf, out_hbm.at[pl.ds(cid * NKEY + sid * KPT, KPT)])


@functools.partial(
    pl.kernel,
    out_type=jax.ShapeDtypeStruct((NC * NKEY,), jnp.float32),
    mesh=_mesh(),
    scratch_types=[
        pltpu.VMEM((EPW,), jnp.int32),
        pltpu.VMEM((EPW,), jnp.int32),
        pltpu.VMEM((NCH, CH), jnp.int32),
        pltpu.VMEM((NCH, CH), jnp.float32),
        pltpu.VMEM((KPT,), jnp.float32),
        pltpu.VMEM_SHARED((NKEY,), jnp.float32),
        pltpu.SemaphoreType.DMA,
    ],
)
def _sc_counts(dst_hbm, rel_hbm, zkey_hbm, ones_hbm, out_hbm, *scratch):
    _sc_counts_body(dst_hbm, rel_hbm, zkey_hbm, ones_hbm, out_hbm, *scratch)


def _sc_w_body(dst_hbm, rel_hbm, c0_hbm, c1_hbm, w_hbm,
               dst_v, rel_v, keys2, g0, g1, wv, sem):
    cid = lax.axis_index("c")
    sid = lax.axis_index("s")
    wid = cid * NS + sid
    base = wid * EPW
    pltpu.sync_copy(dst_hbm.at[pl.ds(base, EPW)], dst_v)
    pltpu.sync_copy(rel_hbm.at[pl.ds(base, EPW)], rel_v)

    def mk(c, carry):
        for j in range(CH // 16):
            o = c * CH + j * 16
            keys2[c, pl.ds(j * 16, 16)] = (
                dst_v[pl.ds(o, 16)] * R + rel_v[pl.ds(o, 16)])
        return carry

    lax.fori_loop(0, NCH, mk, 0)

    def fire(c, carry):
        pltpu.async_copy(c0_hbm.at[keys2.at[c]],
                         g0.at[pl.ds(c * CH, CH)], sem)
        pltpu.async_copy(c1_hbm.at[keys2.at[c]],
                         g1.at[pl.ds(c * CH, CH)], sem)
        return carry

    lax.fori_loop(0, NCH, fire, 0)

    def drain(c, carry):
        pltpu.make_async_copy(c0_hbm.at[keys2.at[c]],
                              g0.at[pl.ds(c * CH, CH)], sem).wait()
        pltpu.make_async_copy(c1_hbm.at[keys2.at[c]],
                              g1.at[pl.ds(c * CH, CH)], sem).wait()
        return carry

    lax.fori_loop(0, NCH, drain, 0)

    def recip(j, carry):
        wv[pl.ds(j * 16, 16)] = 1.0 / (g0[pl.ds(j * 16, 16)]
                                       + g1[pl.ds(j * 16, 16)])
        return carry

    lax.fori_loop(0, EPW // 16, recip, 0)
    pltpu.sync_copy(wv, w_hbm.at[pl.ds(base, EPW)])


@functools.partial(
    pl.kernel,
    out_type=jax.ShapeDtypeStruct((E,), jnp.float32),
    mesh=_mesh(),
    scratch_types=[
        pltpu.VMEM((EPW,), jnp.int32),
        pltpu.VMEM((EPW,), jnp.int32),
        pltpu.VMEM((NCH, CH), jnp.int32),
        pltpu.VMEM((EPW,), jnp.float32),
        pltpu.VMEM((EPW,), jnp.float32),
        pltpu.VMEM((EPW,), jnp.float32),
        pltpu.SemaphoreType.DMA,
    ],
)
def _sc_w(dst_hbm, rel_hbm, c0_hbm, c1_hbm, w_hbm, *scratch):
    _sc_w_body(dst_hbm, rel_hbm, c0_hbm, c1_hbm, w_hbm, *scratch)


HH = HID // 2


def _sc_layer_body(xw_hbm, src_hbm, rel_hbm, dst_hbm, w_hbm, zrow_hbm, out_hbm,
                   src_v, rel_v, dst_v, g2, g2f, d2, wv, rows, agg_sp, sem):
    cid = lax.axis_index("c")
    sid = lax.axis_index("s")
    wid = cid * NS + sid
    base = wid * EPW
    pltpu.sync_copy(src_hbm.at[pl.ds(base, EPW)], src_v)
    pltpu.sync_copy(rel_hbm.at[pl.ds(base, EPW)], rel_v)
    pltpu.sync_copy(dst_hbm.at[pl.ds(base, EPW)], dst_v)
    pltpu.sync_copy(w_hbm.at[pl.ds(base, EPW)], wv)

    def mk(c, carry):
        for j in range(CH // 16):
            o = c * CH + j * 16
            g2[c, pl.ds(j * 16, 16)] = (
                src_v[pl.ds(o, 16)] * (2 * R) + rel_v[pl.ds(o, 16)] * 2)
            d2[c, pl.ds(j * 16, 16)] = dst_v[pl.ds(o, 16)]
        return carry

    lax.fori_loop(0, NCH, mk, 0)
    pltpu.sync_copy(zrow_hbm, rows)

    for f in (0, 1):
        def zloop(j, carry):
            ck = sid + j * NS

            @pl.when(ck < N // CH)
            def _():
                pltpu.sync_copy(rows, agg_sp.at[pl.ds(ck * CH, CH)])

            return carry

        lax.fori_loop(0, N // CH // NS + 1, zloop, 0)

        def mkf(c, carry):
            for j in range(CH // 16):
                g2f[c, pl.ds(j * 16, 16)] = g2[c, pl.ds(j * 16, 16)] + f
            return carry

        lax.fori_loop(0, NCH, mkf, 0)
        plsc.subcore_barrier()

        def work(c, carry):
            pltpu.async_copy(xw_hbm.at[g2f.at[c]], rows, sem).wait()
            for t in range(CH // 16):
                wl = wv[pl.ds(c * CH + t * 16, 16)]
                for e16 in range(16):
                    e = t * 16 + e16
                    ws = wl[e16]
                    for q in range(HH // 16):
                        rows[e, pl.ds(q * 16, 16)] = (
                            rows[e, pl.ds(q * 16, 16)] * ws)
            pltpu.sync_copy(rows, agg_sp.at[d2.at[c]], add=True)
            return carry

        lax.fori_loop(0, NCH, work, 0)
        plsc.subcore_barrier()

        def rloop(j, carry):
            ck = sid + j * NS

            @pl.when(ck < N // CH)
            def _():
                pltpu.sync_copy(agg_sp.at[pl.ds(ck * CH, CH)], rows)
                pltpu.sync_copy(
                    rows,
                    out_hbm.at[pl.ds(f * NC * N + cid * N + ck * CH, CH)])

            return carry

        lax.fori_loop(0, N // CH // NS + 1, rloop, 0)
        if f == 0:
            plsc.subcore_barrier()
        pltpu.sync_copy(zrow_hbm, rows)


@functools.partial(
    pl.kernel,
    out_type=jax.ShapeDtypeStruct((2 * NC * N, HH), jnp.float32),
    mesh=_mesh(),
    compiler_params=pltpu.CompilerParams(use_tc_tiling_on_sc=False),
    scratch_types=[
        pltpu.VMEM((EPW,), jnp.int32),
        pltpu.VMEM((EPW,), jnp.int32),
        pltpu.VMEM((EPW,), jnp.int32),
        pltpu.VMEM((NCH, CH), jnp.int32),
        pltpu.VMEM((NCH, CH), jnp.int32),
        pltpu.VMEM((NCH, CH), jnp.int32),
        pltpu.VMEM((EPW,), jnp.float32),
        pltpu.VMEM((CH, HH), jnp.float32),
        pltpu.VMEM_SHARED((N, HH), jnp.float32),
        pltpu.SemaphoreType.DMA,
    ],
)
def _sc_layer(xw_hbm, src_hbm, rel_hbm, dst_hbm, w_hbm, zrow_hbm, out_hbm,
              *scratch):
    _sc_layer_body(xw_hbm, src_hbm, rel_hbm, dst_hbm, w_hbm, zrow_hbm, out_hbm,
                   *scratch)


def _kw_body(comp1_ref, b1_ref, comp2_ref, b2_ref, w1_ref, w2_ref):
    for cref, bref, oref in ((comp1_ref, b1_ref, w1_ref),
                             (comp2_ref, b2_ref, w2_ref)):
        for r in range(R):
            acc = cref[r, 0] * bref[0]
            for b in range(1, NBASE):
                acc = acc + cref[r, b] * bref[b]
            oref[:, r * HID:(r + 1) * HID] = acc


def _kw(comp1, bases1, comp2, bases2):
    return pl.pallas_call(
        _kw_body,
        in_specs=[
            pl.BlockSpec(memory_space=pltpu.SMEM),
            pl.BlockSpec(memory_space=pltpu.MemorySpace.VMEM),
            pl.BlockSpec(memory_space=pltpu.SMEM),
            pl.BlockSpec(memory_space=pltpu.MemorySpace.VMEM),
        ],
        out_specs=[pl.BlockSpec(memory_space=pltpu.MemorySpace.VMEM),
                   pl.BlockSpec(memory_space=pltpu.MemorySpace.VMEM)],
        out_shape=[jax.ShapeDtypeStruct((CIN, R * HID), jnp.float32),
                   jax.ShapeDtypeStruct((HID, R * HID), jnp.float32)],
    )(comp1, bases1, comp2, bases2)


def _dense_body(x_ref, w_ref, o_ref):
    o_ref[...] = jnp.dot(x_ref[...], w_ref[...],
                         preferred_element_type=jnp.float32)


def _dense(x, w_all):
    return pl.pallas_call(
        _dense_body,
        grid=(NBLK,),
        in_specs=[
            pl.BlockSpec((BN, CIN), lambda i: (i, 0)),
            pl.BlockSpec((CIN, R * HID), lambda i: (0, 0)),
        ],
        out_specs=pl.BlockSpec((BN, R * HID), lambda i: (i, 0)),
        out_shape=jax.ShapeDtypeStruct((N, R * HID), jnp.float32),
    )(x, w_all)


def _comb_body(p_ref, x_ref, root_ref,
               bias_ref, w2_ref, h_ref, o_ref):
    agg = jnp.concatenate([p_ref[0] + p_ref[1],
                           p_ref[2] + p_ref[3]], axis=1)
    h = (agg
         + jnp.dot(x_ref[...], root_ref[...],
                   preferred_element_type=jnp.float32)
         + bias_ref[...])
    h = jnp.maximum(h, 0.0)
    h_ref[...] = h
    o_ref[...] = jnp.dot(h, w2_ref[...], preferred_element_type=jnp.float32)


def _comb(p4, x, root, bias_row, w2_all):
    return pl.pallas_call(
        _comb_body,
        grid=(NBLK,),
        in_specs=[
            pl.BlockSpec((4, BN, HH), lambda i: (0, i, 0)),
            pl.BlockSpec((BN, CIN), lambda i: (i, 0)),
            pl.BlockSpec((CIN, HID), lambda i: (0, 0)),
            pl.BlockSpec((1, HID), lambda i: (0, 0)),
            pl.BlockSpec((HID, R * HID), lambda i: (0, 0)),
        ],
        out_specs=[pl.BlockSpec((BN, HID), lambda i: (i, 0)),
                   pl.BlockSpec((BN, R * HID), lambda i: (i, 0))],
        out_shape=[jax.ShapeDtypeStruct((N, HID), jnp.float32),
                   jax.ShapeDtypeStruct((N, R * HID), jnp.float32)],
    )(p4, x, root, bias_row, w2_all)


def _final_body(q_ref, h1_ref, root_ref,
                bias_ref, batch_ref, wc_ref, bc_ref, o_ref, acc_ref, cnt_ref):
    i = pl.program_id(0)

    @pl.when(i == 0)
    def _init():
        acc_ref[...] = jnp.zeros_like(acc_ref)
        cnt_ref[...] = jnp.zeros_like(cnt_ref)

    agg = jnp.concatenate([q_ref[0] + q_ref[1],
                           q_ref[2] + q_ref[3]], axis=1)
    h2 = (agg
          + jnp.dot(h1_ref[...], root_ref[...],
                    preferred_element_type=jnp.float32)
          + bias_ref[...])
    h2 = jnp.maximum(h2, 0.0)
    b = batch_ref[0, 0, :]
    gi = lax.broadcasted_iota(jnp.int32, (G, BN), 0)
    mask = (gi == b[None, :]).astype(jnp.float32)
    acc_ref[...] += jnp.dot(mask, h2, preferred_element_type=jnp.float32)
    cnt_ref[...] += jnp.sum(mask, axis=1, keepdims=True)

    @pl.when(i == NBLK - 1)
    def _fin():
        pooled = acc_ref[...] / jnp.maximum(cnt_ref[...], 1.0)
        o_ref[...] = (jnp.dot(pooled, wc_ref[...],
                              preferred_element_type=jnp.float32)
                      + bc_ref[...])


def _final(q4, h1, root2, bias2_row, batch3, Wc, bc_row):
    return pl.pallas_call(
        _final_body,
        grid=(NBLK,),
        in_specs=[
            pl.BlockSpec((4, BN, HH), lambda i: (0, i, 0)),
            pl.BlockSpec((BN, HID), lambda i: (i, 0)),
            pl.BlockSpec((HID, HID), lambda i: (0, 0)),
            pl.BlockSpec((1, HID), lambda i: (0, 0)),
            pl.BlockSpec((1, 1, BN), lambda i: (i, 0, 0)),
            pl.BlockSpec((HID, OUT), lambda i: (0, 0)),
            pl.BlockSpec((1, OUT), lambda i: (0, 0)),
        ],
        out_specs=pl.BlockSpec((G, OUT), lambda i: (0, 0)),
        out_shape=jax.ShapeDtypeStruct((G, OUT), jnp.float32),
        scratch_shapes=[pltpu.VMEM((G, HID), jnp.float32),
                        pltpu.VMEM((G, 1), jnp.float32)],
    )(q4, h1, root2, bias2_row, batch3, Wc, bc_row)


def kernel(x, edge_index, edge_type, batch, bases1, comp1, root1, bias1,
           bases2, comp2, root2, bias2, Wc, bc):
    src = edge_index[0]
    dst = edge_index[1]
    rel = edge_type
    batch3 = batch.reshape(NBLK, 1, BN)
    zkey = jnp.zeros((KPT,), jnp.float32)
    zrow = jnp.zeros((CH, HH), jnp.float32)
    bias1r = bias1.reshape(1, HID)
    bias2r = bias2.reshape(1, HID)
    bcr = bc.reshape(1, OUT)

    w1a, w2a = _kw(comp1, bases1, comp2, bases2)
    ones2 = jnp.ones((NCH, CH), jnp.float32)
    cnts = _sc_counts(dst, rel, zkey, ones2)
    wvec = _sc_w(dst, rel, cnts[:NKEY], cnts[NKEY:])
    xw1 = _dense(x, w1a).reshape(2 * NKEY, HH)
    a1 = _sc_layer(xw1, src, rel, dst, wvec, zrow).reshape(4, N, HH)
    h1, xw2 = _comb(a1, x, root1, bias1r, w2a)
    a2 = _sc_layer(xw2.reshape(2 * NKEY, HH), src, rel, dst, wvec,
                   zrow).reshape(4, N, HH)
    return _final(a2, h1, root2, bias2r, batch3, Wc, bcr)


# speedup vs baseline: 2.5318x; 1.5709x over previous
import functools

import jax
import jax.numpy as jnp
from jax import lax
from jax.experimental import pallas as pl
from jax.experimental.pallas import tpu as pltpu
from jax.experimental.pallas import tpu_sc as plsc

N = 10000
E = 320000
CIN = 128
HID = 128
OUT = 10
R = 8
NBASE = 4
G = 64

NC = 2
NS = 16
NW = NC * NS
EPW = E // NW
CH = 80
NCH = EPW // CH
NKEY = N * R
KPT = NKEY // NS


BN = 1000
NBLK = N // BN


def _mesh():
    return plsc.VectorSubcoreMesh(core_axis_name="c", subcore_axis_name="s")


def _sc_counts_body(dst_hbm, rel_hbm, zkey_hbm, ones_hbm, out_hbm,
                    dst_v, rel_v, keys2, ones_v, zbuf, cnt_sp, sem):
    cid = lax.axis_index("c")
    sid = lax.axis_index("s")
    wid = cid * NS + sid
    base = wid * EPW
    pltpu.sync_copy(zkey_hbm, zbuf)
    pltpu.sync_copy(zbuf, cnt_sp.at[pl.ds(sid * KPT, KPT)])
    pltpu.sync_copy(dst_hbm.at[pl.ds(base, EPW)], dst_v)
    pltpu.sync_copy(rel_hbm.at[pl.ds(base, EPW)], rel_v)
    pltpu.sync_copy(ones_hbm, ones_v)

    def mk(c, carry):
        for j in range(CH // 16):
            o = c * CH + j * 16
            keys2[c, pl.ds(j * 16, 16)] = (
                dst_v[pl.ds(o, 16)] * R + rel_v[pl.ds(o, 16)])
        return carry

    lax.fori_loop(0, NCH, mk, 0)
    plsc.subcore_barrier()

    def scat(c, carry):
        pltpu.async_copy(ones_v.at[c], cnt_sp.at[keys2.at[c]], sem, add=True)
        return carry

    lax.fori_loop(0, NCH, scat, 0)

    def drain(c, carry):
        pltpu.make_async_copy(ones_v.at[c], cnt_sp.at[keys2.at[c]], sem).wait()
        return carry

    lax.fori_loop(0, NCH, drain, 0)
    plsc.subcore_barrier()
    pltpu.sync_copy(cnt_sp.at[pl.ds(sid * KPT, KPT)], zbuf)
    pltpu.sync_copy(zbuf, out_hbm.at[pl.ds(cid * NKEY + sid * KPT, KPT)])


@functools.partial(
    pl.kernel,
    out_type=jax.ShapeDtypeStruct((NC * NKEY,), jnp.float32),
    mesh=_mesh(),
    scratch_types=[
        pltpu.VMEM((EPW,), jnp.int32),
        pltpu.VMEM((EPW,), jnp.int32),
        pltpu.VMEM((NCH, CH), jnp.int32),
        pltpu.VMEM((NCH, CH), jnp.float32),
        pltpu.VMEM((KPT,), jnp.float32),
        pltpu.VMEM_SHARED((NKEY,), jnp.float32),
        pltpu.SemaphoreType.DMA,
    ],
)
def _sc_counts(dst_hbm, rel_hbm, zkey_hbm, ones_hbm, out_hbm, *scratch):
    _sc_counts_body(dst_hbm, rel_hbm, zkey_hbm, ones_hbm, out_hbm, *scratch)


def _sc_w_body(dst_hbm, rel_hbm, c0_hbm, c1_hbm, w_hbm,
               dst_v, rel_v, keys2, g0, g1, wv, sem):
    cid = lax.axis_index("c")
    sid = lax.axis_index("s")
    wid = cid * NS + sid
    base = wid * EPW
    pltpu.sync_copy(dst_hbm.at[pl.ds(base, EPW)], dst_v)
    pltpu.sync_copy(rel_hbm.at[pl.ds(base, EPW)], rel_v)

    def mk(c, carry):
        for j in range(CH // 16):
            o = c * CH + j * 16
            keys2[c, pl.ds(j * 16, 16)] = (
                dst_v[pl.ds(o, 16)] * R + rel_v[pl.ds(o, 16)])
        return carry

    lax.fori_loop(0, NCH, mk, 0)

    def fire(c, carry):
        pltpu.async_copy(c0_hbm.at[keys2.at[c]],
                         g0.at[pl.ds(c * CH, CH)], sem)
        pltpu.async_copy(c1_hbm.at[keys2.at[c]],
                         g1.at[pl.ds(c * CH, CH)], sem)
        return carry

    lax.fori_loop(0, NCH, fire, 0)

    def drain(c, carry):
        pltpu.make_async_copy(c0_hbm.at[keys2.at[c]],
                              g0.at[pl.ds(c * CH, CH)], sem).wait()
        pltpu.make_async_copy(c1_hbm.at[keys2.at[c]],
                              g1.at[pl.ds(c * CH, CH)], sem).wait()
        return carry

    lax.fori_loop(0, NCH, drain, 0)

    def recip(j, carry):
        wv[pl.ds(j * 16, 16)] = 1.0 / (g0[pl.ds(j * 16, 16)]
                                       + g1[pl.ds(j * 16, 16)])
        return carry

    lax.fori_loop(0, EPW // 16, recip, 0)
    pltpu.sync_copy(wv, w_hbm.at[pl.ds(base, EPW)])


@functools.partial(
    pl.kernel,
    out_type=jax.ShapeDtypeStruct((E,), jnp.float32),
    mesh=_mesh(),
    scratch_types=[
        pltpu.VMEM((EPW,), jnp.int32),
        pltpu.VMEM((EPW,), jnp.int32),
        pltpu.VMEM((NCH, CH), jnp.int32),
        pltpu.VMEM((EPW,), jnp.float32),
        pltpu.VMEM((EPW,), jnp.float32),
        pltpu.VMEM((EPW,), jnp.float32),
        pltpu.SemaphoreType.DMA,
    ],
)
def _sc_w(dst_hbm, rel_hbm, c0_hbm, c1_hbm, w_hbm, *scratch):
    _sc_w_body(dst_hbm, rel_hbm, c0_hbm, c1_hbm, w_hbm, *scratch)


HH = HID // 2


RD = 4


def _sc_layer_body(xw_hbm, src_hbm, rel_hbm, dst_hbm, w_hbm, zrow_hbm, out_hbm,
                   src_v, rel_v, dst_v, g2, g2f, d2, wv, rows, rows1, rows2,
                   rows3, agg_sp, sem, sem1, sem2, sem3):
    cid = lax.axis_index("c")
    sid = lax.axis_index("s")
    wid = cid * NS + sid
    base = wid * EPW
    pltpu.sync_copy(src_hbm.at[pl.ds(base, EPW)], src_v)
    pltpu.sync_copy(rel_hbm.at[pl.ds(base, EPW)], rel_v)
    pltpu.sync_copy(dst_hbm.at[pl.ds(base, EPW)], dst_v)
    pltpu.sync_copy(w_hbm.at[pl.ds(base, EPW)], wv)

    def mk(c, carry):
        for j in range(CH // 16):
            o = c * CH + j * 16
            g2[c, pl.ds(j * 16, 16)] = (
                src_v[pl.ds(o, 16)] * (2 * R) + rel_v[pl.ds(o, 16)] * 2)
            d2[c, pl.ds(j * 16, 16)] = dst_v[pl.ds(o, 16)]
        return carry

    lax.fori_loop(0, NCH, mk, 0)
    pltpu.sync_copy(zrow_hbm, rows)

    for f in (0, 1):
        def zloop(j, carry):
            ck = sid + j * NS

            @pl.when(ck < N // CH)
            def _():
                pltpu.sync_copy(rows, agg_sp.at[pl.ds(ck * CH, CH)])

            return carry

        lax.fori_loop(0, N // CH // NS + 1, zloop, 0)

        def mkf(c, carry):
            for j in range(CH // 16):
                g2f[c, pl.ds(j * 16, 16)] = g2[c, pl.ds(j * 16, 16)] + f
            return carry

        lax.fori_loop(0, NCH, mkf, 0)
        plsc.subcore_barrier()

        def do_scale(buf, c):
            for t in range(CH // 16):
                wl = wv[pl.ds(c * CH + t * 16, 16)]
                for e16 in range(16):
                    e = t * 16 + e16
                    ws = wl[e16]
                    for q in range(HH // 16):
                        buf[e, pl.ds(q * 16, 16)] = (
                            buf[e, pl.ds(q * 16, 16)] * ws)

        ring = (rows, rows1, rows2, rows3)
        sems = (sem, sem1, sem2, sem3)
        for k in range(RD):
            pltpu.async_copy(xw_hbm.at[g2f.at[k]], ring[k], sems[k])

        def work4(i, carry):
            for k in range(RD):
                c = RD * i + k
                pltpu.make_async_copy(xw_hbm.at[g2f.at[c]], ring[k],
                                      sems[k]).wait()
                do_scale(ring[k], c)
                pltpu.sync_copy(ring[k], agg_sp.at[d2.at[c]], add=True)
                cn = lax.rem(c + RD, NCH)
                pltpu.async_copy(xw_hbm.at[g2f.at[cn]], ring[k], sems[k])
            return carry

        lax.fori_loop(0, NCH // RD, work4, 0)
        clast = NCH - 1
        pltpu.make_async_copy(xw_hbm.at[g2f.at[clast]], ring[0],
                              sems[0]).wait()
        do_scale(ring[0], clast)
        pltpu.sync_copy(ring[0], agg_sp.at[d2.at[clast]], add=True)
        for k in range(1, RD):
            pltpu.make_async_copy(xw_hbm.at[g2f.at[k]], ring[k],
                                  sems[k]).wait()
        plsc.subcore_barrier()

        def rloop(j, carry):
            ck = sid + j * NS

            @pl.when(ck < N // CH)
            def _():
                pltpu.sync_copy(agg_sp.at[pl.ds(ck * CH, CH)], rows)
                pltpu.sync_copy(
                    rows,
                    out_hbm.at[pl.ds(f * NC * N + cid * N + ck * CH, CH)])

            return carry

        lax.fori_loop(0, N // CH // NS + 1, rloop, 0)
        if f == 0:
            plsc.subcore_barrier()
        pltpu.sync_copy(zrow_hbm, rows)


@functools.partial(
    pl.kernel,
    out_type=jax.ShapeDtypeStruct((2 * NC * N, HH), jnp.float32),
    mesh=_mesh(),
    compiler_params=pltpu.CompilerParams(use_tc_tiling_on_sc=False),
    scratch_types=[
        pltpu.VMEM((EPW,), jnp.int32),
        pltpu.VMEM((EPW,), jnp.int32),
        pltpu.VMEM((EPW,), jnp.int32),
        pltpu.VMEM((NCH, CH), jnp.int32),
        pltpu.VMEM((NCH, CH), jnp.int32),
        pltpu.VMEM((NCH, CH), jnp.int32),
        pltpu.VMEM((EPW,), jnp.float32),
        pltpu.VMEM((CH, HH), jnp.float32),
        pltpu.VMEM((CH, HH), jnp.float32),
        pltpu.VMEM((CH, HH), jnp.float32),
        pltpu.VMEM((CH, HH), jnp.float32),
        pltpu.VMEM_SHARED((N, HH), jnp.float32),
        pltpu.SemaphoreType.DMA,
        pltpu.SemaphoreType.DMA,
        pltpu.SemaphoreType.DMA,
        pltpu.SemaphoreType.DMA,
    ],
)
def _sc_layer(xw_hbm, src_hbm, rel_hbm, dst_hbm, w_hbm, zrow_hbm, out_hbm,
              *scratch):
    _sc_layer_body(xw_hbm, src_hbm, rel_hbm, dst_hbm, w_hbm, zrow_hbm, out_hbm,
                   *scratch)


def _kw_body(comp1_ref, b1_ref, comp2_ref, b2_ref, w1_ref, w2_ref):
    for cref, bref, oref in ((comp1_ref, b1_ref, w1_ref),
                             (comp2_ref, b2_ref, w2_ref)):
        for r in range(R):
            acc = cref[r, 0] * bref[0]
            for b in range(1, NBASE):
                acc = acc + cref[r, b] * bref[b]
            oref[:, r * HID:(r + 1) * HID] = acc


def _kw(comp1, bases1, comp2, bases2):
    return pl.pallas_call(
        _kw_body,
        in_specs=[
            pl.BlockSpec(memory_space=pltpu.SMEM),
            pl.BlockSpec(memory_space=pltpu.MemorySpace.VMEM),
            pl.BlockSpec(memory_space=pltpu.SMEM),
            pl.BlockSpec(memory_space=pltpu.MemorySpace.VMEM),
        ],
        out_specs=[pl.BlockSpec(memory_space=pltpu.MemorySpace.VMEM),
                   pl.BlockSpec(memory_space=pltpu.MemorySpace.VMEM)],
        out_shape=[jax.ShapeDtypeStruct((CIN, R * HID), jnp.float32),
                   jax.ShapeDtypeStruct((HID, R * HID), jnp.float32)],
    )(comp1, bases1, comp2, bases2)


def _dense_body(x_ref, w_ref, o_ref):
    o_ref[...] = jnp.dot(x_ref[...], w_ref[...],
                         preferred_element_type=jnp.float32)


def _dense(x, w_all):
    return pl.pallas_call(
        _dense_body,
        grid=(NBLK,),
        in_specs=[
            pl.BlockSpec((BN, CIN), lambda i: (i, 0)),
            pl.BlockSpec((CIN, R * HID), lambda i: (0, 0)),
        ],
        out_specs=pl.BlockSpec((BN, R * HID), lambda i: (i, 0)),
        out_shape=jax.ShapeDtypeStruct((N, R * HID), jnp.float32),
    )(x, w_all)


def _comb_body(p_ref, x_ref, root_ref,
               bias_ref, w2_ref, h_ref, o_ref):
    agg = jnp.concatenate([p_ref[0] + p_ref[1],
                           p_ref[2] + p_ref[3]], axis=1)
    h = (agg
         + jnp.dot(x_ref[...], root_ref[...],
                   preferred_element_type=jnp.float32)
         + bias_ref[...])
    h = jnp.maximum(h, 0.0)
    h_ref[...] = h
    o_ref[...] = jnp.dot(h, w2_ref[...], preferred_element_type=jnp.float32)


def _comb(p4, x, root, bias_row, w2_all):
    return pl.pallas_call(
        _comb_body,
        grid=(NBLK,),
        in_specs=[
            pl.BlockSpec((4, BN, HH), lambda i: (0, i, 0)),
            pl.BlockSpec((BN, CIN), lambda i: (i, 0)),
            pl.BlockSpec((CIN, HID), lambda i: (0, 0)),
            pl.BlockSpec((1, HID), lambda i: (0, 0)),
            pl.BlockSpec((HID, R * HID), lambda i: (0, 0)),
        ],
        out_specs=[pl.BlockSpec((BN, HID), lambda i: (i, 0)),
                   pl.BlockSpec((BN, R * HID), lambda i: (i, 0))],
        out_shape=[jax.ShapeDtypeStruct((N, HID), jnp.float32),
                   jax.ShapeDtypeStruct((N, R * HID), jnp.float32)],
    )(p4, x, root, bias_row, w2_all)


def _final_body(q_ref, h1_ref, root_ref,
                bias_ref, batch_ref, wc_ref, bc_ref, o_ref, acc_ref, cnt_ref):
    i = pl.program_id(0)

    @pl.when(i == 0)
    def _init():
        acc_ref[...] = jnp.zeros_like(acc_ref)
        cnt_ref[...] = jnp.zeros_like(cnt_ref)

    agg = jnp.concatenate([q_ref[0] + q_ref[1],
                           q_ref[2] + q_ref[3]], axis=1)
    h2 = (agg
          + jnp.dot(h1_ref[...], root_ref[...],
                    preferred_element_type=jnp.float32)
          + bias_ref[...])
    h2 = jnp.maximum(h2, 0.0)
    b = batch_ref[0, 0, :]
    gi = lax.broadcasted_iota(jnp.int32, (G, BN), 0)
    mask = (gi == b[None, :]).astype(jnp.float32)
    acc_ref[...] += jnp.dot(mask, h2, preferred_element_type=jnp.float32)
    cnt_ref[...] += jnp.sum(mask, axis=1, keepdims=True)

    @pl.when(i == NBLK - 1)
    def _fin():
        pooled = acc_ref[...] / jnp.maximum(cnt_ref[...], 1.0)
        o_ref[...] = (jnp.dot(pooled, wc_ref[...],
                              preferred_element_type=jnp.float32)
                      + bc_ref[...])


def _final(q4, h1, root2, bias2_row, batch3, Wc, bc_row):
    return pl.pallas_call(
        _final_body,
        grid=(NBLK,),
        in_specs=[
            pl.BlockSpec((4, BN, HH), lambda i: (0, i, 0)),
            pl.BlockSpec((BN, HID), lambda i: (i, 0)),
            pl.BlockSpec((HID, HID), lambda i: (0, 0)),
            pl.BlockSpec((1, HID), lambda i: (0, 0)),
            pl.BlockSpec((1, 1, BN), lambda i: (i, 0, 0)),
            pl.BlockSpec((HID, OUT), lambda i: (0, 0)),
            pl.BlockSpec((1, OUT), lambda i: (0, 0)),
        ],
        out_specs=pl.BlockSpec((G, OUT), lambda i: (0, 0)),
        out_shape=jax.ShapeDtypeStruct((G, OUT), jnp.float32),
        scratch_shapes=[pltpu.VMEM((G, HID), jnp.float32),
                        pltpu.VMEM((G, 1), jnp.float32)],
    )(q4, h1, root2, bias2_row, batch3, Wc, bc_row)


def kernel(x, edge_index, edge_type, batch, bases1, comp1, root1, bias1,
           bases2, comp2, root2, bias2, Wc, bc):
    src = edge_index[0]
    dst = edge_index[1]
    rel = edge_type
    batch3 = batch.reshape(NBLK, 1, BN)
    zkey = jnp.zeros((KPT,), jnp.float32)
    zrow = jnp.zeros((CH, HH), jnp.float32)
    bias1r = bias1.reshape(1, HID)
    bias2r = bias2.reshape(1, HID)
    bcr = bc.reshape(1, OUT)

    w1a, w2a = _kw(comp1, bases1, comp2, bases2)
    ones2 = jnp.ones((NCH, CH), jnp.float32)
    cnts = _sc_counts(dst, rel, zkey, ones2)
    wvec = _sc_w(dst, rel, cnts[:NKEY], cnts[NKEY:])
    xw1 = _dense(x, w1a).reshape(2 * NKEY, HH)
    a1 = _sc_layer(xw1, src, rel, dst, wvec, zrow).reshape(4, N, HH)
    h1, xw2 = _comb(a1, x, root1, bias1r, w2a)
    a2 = _sc_layer(xw2.reshape(2 * NKEY, HH), src, rel, dst, wvec,
                   zrow).reshape(4, N, HH)
    return _final(a2, h1, root2, bias2r, batch3, Wc, bcr)
